# baseline (speedup 1.0000x reference)
import functools

import jax
import jax.numpy as jnp
from jax import lax
from jax.experimental import pallas as pl
from jax.experimental.pallas import tpu as pltpu
from jax.experimental.pallas import tpu_sc as plsc

N_NODES = 10000
E = 320000
D = 128
L = 16
NC, NS = 2, 16
NW = NC * NS
EPW = E // NW
CHUNK = 80
NCHUNK = EPW // CHUNK
NGROUP = CHUNK // L


def _pack_rows(h):
    bl = lax.bitcast_convert_type(h[:, : D // 2], jnp.int32)
    bl = bl + jnp.int32(0x7FFF) + (lax.shift_right_logical(bl, 16) & jnp.int32(1))
    lo = lax.shift_right_logical(bl, 16)
    hi = lax.bitcast_convert_type(h[:, D // 2 :], jnp.int32) & jnp.int32(-65536)
    return lo | hi


def _precompute_body(zu_ref, zi_ref, w1_ref, b1_ref, hu_ref, hi_ref):
    cdims = (((1,), (1,)), ((), ()))
    hu = lax.dot_general(zu_ref[...], w1_ref[:, :D], cdims,
                         preferred_element_type=jnp.float32) + b1_ref[...]
    hv = lax.dot_general(zi_ref[...], w1_ref[:, D:], cdims,
                         preferred_element_type=jnp.float32)
    hu_ref[...] = _pack_rows(hu)
    hi_ref[...] = _pack_rows(hv)


def _precompute_tables(z_user, z_item, w1, b1):
    blk = N_NODES
    grid = (1,)
    return pl.pallas_call(
        _precompute_body,
        grid=grid,
        in_specs=[
            pl.BlockSpec((blk, D), lambda i: (i, 0)),
            pl.BlockSpec((blk, D), lambda i: (i, 0)),
            pl.BlockSpec((D, 2 * D), lambda i: (0, 0)),
            pl.BlockSpec((1, D), lambda i: (0, 0)),
        ],
        out_specs=[
            pl.BlockSpec((blk, D // 2), lambda i: (i, 0)),
            pl.BlockSpec((blk, D // 2), lambda i: (i, 0)),
        ],
        out_shape=[
            jax.ShapeDtypeStruct((N_NODES, D // 2), jnp.int32),
            jax.ShapeDtypeStruct((N_NODES, D // 2), jnp.int32),
        ],
    )(z_user, z_item, w1, b1)


def _edge_body(hu_hbm, hi_hbm, row_hbm, col_hbm, w2_hbm, b2_hbm, out_hbm,
               idx_u, idx_i, rows_u, rows_i, out_v, w2_v, b2_v,
               sem_u0, sem_u1, sem_i0, sem_i1):
    sem_u = (sem_u0, sem_u1)
    sem_i = (sem_i0, sem_i1)
    wid = lax.axis_index("s") * NC + lax.axis_index("c")

    pltpu.sync_copy(row_hbm.at[pl.ds(wid * EPW, EPW)], idx_u)
    pltpu.sync_copy(col_hbm.at[pl.ds(wid * EPW, EPW)], idx_i)
    pltpu.sync_copy(w2_hbm, w2_v)
    pltpu.sync_copy(b2_hbm, b2_v)

    w2v = [w2_v[pl.ds(j * L, L)] for j in range(D // L)]
    b2v = b2_v[...]
    lane = lax.broadcasted_iota(jnp.int32, (L,), 0)
    zero = jnp.zeros((L,), jnp.float32)
    masks = [lane == k for k in range(L)]
    perms = [lax.bitwise_xor(lane, jnp.int32(s)) for s in (1, 2, 4, 8)]

    def lane_sum(x):
        for p in perms:
            x = x + x.at[p].get(mode="promise_in_bounds")
        return x

    def copies(c, slot):
        return (
            pltpu.make_async_copy(hu_hbm.at[idx_u.at[pl.ds(c * CHUNK, CHUNK)]],
                                  rows_u.at[slot], sem_u[slot]),
            pltpu.make_async_copy(hi_hbm.at[idx_i.at[pl.ds(c * CHUNK, CHUNK)]],
                                  rows_i.at[slot], sem_i[slot]),
        )

    def start(c, slot):
        cu, ci = copies(c, slot)
        cu.start()
        ci.start()

    def wait(c, slot):
        cu, ci = copies(c, slot)
        cu.wait()
        ci.wait()

    def compute(c, slot):
        @plsc.parallel_loop(0, NGROUP)
        def group(g):
            res = zero
            for k in range(L):
                b = g * L + k
                acc = zero
                for j in range(D // (2 * L)):
                    ui = rows_u[slot, b, pl.ds(j * L, L)]
                    vi = rows_i[slot, b, pl.ds(j * L, L)]
                    ua = lax.bitcast_convert_type(lax.shift_left(ui, 16), jnp.float32)
                    ub = lax.bitcast_convert_type(ui & jnp.int32(-65536), jnp.float32)
                    va = lax.bitcast_convert_type(lax.shift_left(vi, 16), jnp.float32)
                    vb = lax.bitcast_convert_type(vi & jnp.int32(-65536), jnp.float32)
                    ra = jnp.maximum(ua + va, 0.0)
                    rb = jnp.maximum(ub + vb, 0.0)
                    acc = acc + ra * w2v[2 * j] + rb * w2v[2 * j + 1]
                res = jnp.where(masks[k], lane_sum(acc), res)
            out_v[pl.ds(c * CHUNK + g * L, L)] = res + b2v

    start(0, 0)

    @pl.loop(0, (NCHUNK + 1) // 2, unroll=1)
    def pair(t):
        c = 2 * t

        @pl.when(c + 1 < NCHUNK)
        def _():
            start(c + 1, 1)

        wait(c, 0)
        compute(c, 0)

        @pl.when(c + 2 < NCHUNK)
        def _():
            start(c + 2, 0)

        @pl.when(c + 1 < NCHUNK)
        def _():
            wait(c + 1, 1)
            compute(c + 1, 1)
    pltpu.sync_copy(out_v, out_hbm.at[pl.ds(wid * EPW, EPW)])


def _edge_decode(hu, hi, row, col, w2, b2):
    mesh = plsc.VectorSubcoreMesh(core_axis_name="c", subcore_axis_name="s")
    k = functools.partial(
        pl.kernel,
        mesh=mesh,
        compiler_params=pltpu.CompilerParams(use_tc_tiling_on_sc=False),
        out_type=jax.ShapeDtypeStruct((E,), jnp.float32),
        scratch_types=[
            pltpu.VMEM((EPW,), jnp.int32),
            pltpu.VMEM((EPW,), jnp.int32),
            pltpu.VMEM((2, CHUNK, D // 2), jnp.int32),
            pltpu.VMEM((2, CHUNK, D // 2), jnp.int32),
            pltpu.VMEM((EPW,), jnp.float32),
            pltpu.VMEM((D,), jnp.float32),
            pltpu.VMEM((L,), jnp.float32),
            pltpu.SemaphoreType.DMA,
            pltpu.SemaphoreType.DMA,
            pltpu.SemaphoreType.DMA,
            pltpu.SemaphoreType.DMA,
        ],
    )(_edge_body)
    return k(hu, hi, row, col, w2, b2)


def kernel(z_user, z_item, edge_label_index, W1, b1, W2, b2):
    hu, hi = _precompute_tables(z_user, z_item, W1, b1.reshape(1, D))

    eli = edge_label_index.astype(jnp.int32)
    row = eli[0]
    col = eli[1]
    w2 = W2.reshape(2, D // (2 * L), L).transpose(1, 0, 2).reshape(D)
    b2v = jnp.broadcast_to(b2.reshape(1), (L,))
    return _edge_decode(hu, hi, row, col, w2, b2v)


# speedup vs baseline: 1.0585x; 1.0585x over previous
import functools

import jax
import jax.numpy as jnp
from jax import lax
from jax.experimental import pallas as pl
from jax.experimental.pallas import tpu as pltpu
from jax.experimental.pallas import tpu_sc as plsc

N_NODES = 10000
E = 320000
D = 128
L = 16
NC, NS = 2, 16
NW = NC * NS
EPW = E // NW
CHUNK = 80
NCHUNK = EPW // CHUNK
NGROUP = CHUNK // L


def _pack_rows(h):
    bl = lax.bitcast_convert_type(h[:, : D // 2], jnp.int32)
    bl = bl + jnp.int32(0x7FFF) + (lax.shift_right_logical(bl, 16) & jnp.int32(1))
    lo = lax.shift_right_logical(bl, 16)
    hi = lax.bitcast_convert_type(h[:, D // 2 :], jnp.int32) & jnp.int32(-65536)
    return lo | hi


def _precompute_body(zu_ref, zi_ref, w1_ref, b1_ref, hu_ref, hi_ref):
    cdims = (((1,), (1,)), ((), ()))
    hu = lax.dot_general(zu_ref[...], w1_ref[:, :D], cdims,
                         preferred_element_type=jnp.float32) + b1_ref[...]
    hv = lax.dot_general(zi_ref[...], w1_ref[:, D:], cdims,
                         preferred_element_type=jnp.float32)
    hu_ref[...] = _pack_rows(hu)
    hi_ref[...] = _pack_rows(hv)


def _precompute_tables(z_user, z_item, w1, b1):
    blk = N_NODES
    grid = (1,)
    return pl.pallas_call(
        _precompute_body,
        grid=grid,
        in_specs=[
            pl.BlockSpec((blk, D), lambda i: (i, 0)),
            pl.BlockSpec((blk, D), lambda i: (i, 0)),
            pl.BlockSpec((D, 2 * D), lambda i: (0, 0)),
            pl.BlockSpec((1, D), lambda i: (0, 0)),
        ],
        out_specs=[
            pl.BlockSpec((blk, D // 2), lambda i: (i, 0)),
            pl.BlockSpec((blk, D // 2), lambda i: (i, 0)),
        ],
        out_shape=[
            jax.ShapeDtypeStruct((N_NODES, D // 2), jnp.int32),
            jax.ShapeDtypeStruct((N_NODES, D // 2), jnp.int32),
        ],
    )(z_user, z_item, w1, b1)


def _edge_body(hu_hbm, hi_hbm, eli_hbm, w2_hbm, b2_hbm, out_hbm,
               idx_u, idx_i, rows_u, rows_i, out_v, w2_v, b2_v,
               sem_u0, sem_u1, sem_i0, sem_i1):
    sem_u = (sem_u0, sem_u1)
    sem_i = (sem_i0, sem_i1)
    wid = lax.axis_index("s") * NC + lax.axis_index("c")

    pltpu.sync_copy(eli_hbm.at[0, pl.ds(wid * EPW, EPW)], idx_u)
    pltpu.sync_copy(eli_hbm.at[1, pl.ds(wid * EPW, EPW)], idx_i)
    pltpu.sync_copy(w2_hbm, w2_v)
    pltpu.sync_copy(b2_hbm, b2_v)

    w2v = [w2_v[pl.ds(j * L, L)] for j in range(D // L)]
    b2v = b2_v[...]
    lane = lax.broadcasted_iota(jnp.int32, (L,), 0)
    zero = jnp.zeros((L,), jnp.float32)
    masks = [lane == k for k in range(L)]
    perms = [lax.bitwise_xor(lane, jnp.int32(s)) for s in (1, 2, 4, 8)]

    def lane_sum(x):
        for p in perms:
            x = x + x.at[p].get(mode="promise_in_bounds")
        return x

    def copies(c, slot):
        return (
            pltpu.make_async_copy(hu_hbm.at[idx_u.at[pl.ds(c * CHUNK, CHUNK)]],
                                  rows_u.at[slot], sem_u[slot]),
            pltpu.make_async_copy(hi_hbm.at[idx_i.at[pl.ds(c * CHUNK, CHUNK)]],
                                  rows_i.at[slot], sem_i[slot]),
        )

    def start(c, slot):
        cu, ci = copies(c, slot)
        cu.start()
        ci.start()

    def wait(c, slot):
        cu, ci = copies(c, slot)
        cu.wait()
        ci.wait()

    def compute(c, slot):
        @pl.loop(0, NGROUP, unroll=1)
        def group(g):
            res = zero
            for k in range(L):
                b = g * L + k
                acc = zero
                for j in range(D // (2 * L)):
                    ui = rows_u[slot, b, pl.ds(j * L, L)]
                    vi = rows_i[slot, b, pl.ds(j * L, L)]
                    ua = lax.bitcast_convert_type(lax.shift_left(ui, 16), jnp.float32)
                    ub = lax.bitcast_convert_type(ui & jnp.int32(-65536), jnp.float32)
                    va = lax.bitcast_convert_type(lax.shift_left(vi, 16), jnp.float32)
                    vb = lax.bitcast_convert_type(vi & jnp.int32(-65536), jnp.float32)
                    ra = jnp.maximum(ua + va, 0.0)
                    rb = jnp.maximum(ub + vb, 0.0)
                    acc = acc + ra * w2v[2 * j] + rb * w2v[2 * j + 1]
                res = jnp.where(masks[k], lane_sum(acc), res)
            out_v[pl.ds(c * CHUNK + g * L, L)] = res + b2v

    start(0, 0)

    @pl.loop(0, (NCHUNK + 1) // 2, unroll=1)
    def pair(t):
        c = 2 * t

        @pl.when(c + 1 < NCHUNK)
        def _():
            start(c + 1, 1)

        wait(c, 0)
        compute(c, 0)

        @pl.when(c + 2 < NCHUNK)
        def _():
            start(c + 2, 0)

        @pl.when(c + 1 < NCHUNK)
        def _():
            wait(c + 1, 1)
            compute(c + 1, 1)
    pltpu.sync_copy(out_v, out_hbm.at[pl.ds(wid * EPW, EPW)])


def _edge_decode(hu, hi, eli, w2, b2):
    mesh = plsc.VectorSubcoreMesh(core_axis_name="c", subcore_axis_name="s")
    k = functools.partial(
        pl.kernel,
        mesh=mesh,
        compiler_params=pltpu.CompilerParams(use_tc_tiling_on_sc=False),
        out_type=jax.ShapeDtypeStruct((E,), jnp.float32),
        scratch_types=[
            pltpu.VMEM((EPW,), jnp.int32),
            pltpu.VMEM((EPW,), jnp.int32),
            pltpu.VMEM((2, CHUNK, D // 2), jnp.int32),
            pltpu.VMEM((2, CHUNK, D // 2), jnp.int32),
            pltpu.VMEM((EPW,), jnp.float32),
            pltpu.VMEM((D,), jnp.float32),
            pltpu.VMEM((L,), jnp.float32),
            pltpu.SemaphoreType.DMA,
            pltpu.SemaphoreType.DMA,
            pltpu.SemaphoreType.DMA,
            pltpu.SemaphoreType.DMA,
        ],
    )(_edge_body)
    return k(hu, hi, eli, w2, b2)


def kernel(z_user, z_item, edge_label_index, W1, b1, W2, b2):
    hu, hi = _precompute_tables(z_user, z_item, W1, b1.reshape(1, D))

    eli = edge_label_index.astype(jnp.int32)
    w2 = W2.reshape(2, D // (2 * L), L).transpose(1, 0, 2).reshape(D)
    b2v = jnp.broadcast_to(b2.reshape(1), (L,))
    return _edge_decode(hu, hi, eli, w2, b2v)
